# baseline (speedup 1.0000x reference)
import functools

import jax
import jax.numpy as jnp
from jax import lax
from jax.experimental import pallas as pl
from jax.experimental.pallas import tpu as pltpu
from jax.experimental.pallas import tpu_sc as plsc

B, C, L = 4, 1024, 8192
K = 128
S = C // K
NW = 32
ROWS_PER_W = (B * K) // NW


def _sc_gather(x_hbm, out_hbm):
    wid = lax.axis_index("s") * 2 + lax.axis_index("c")
    b = wid // (K // ROWS_PER_W)
    k0 = (wid % (K // ROWS_PER_W)) * ROWS_PER_W
    pltpu.sync_copy(
        x_hbm.at[pl.ds(b, 1), pl.ds(k0, ROWS_PER_W), pl.ds(0, 1), :],
        out_hbm.at[pl.ds(b, 1), pl.ds(k0, ROWS_PER_W), pl.ds(0, 1), :],
    )


_gather_call = functools.partial(
    pl.kernel,
    out_type=jax.ShapeDtypeStruct((B, K, 1, L), jnp.float32),
    mesh=plsc.VectorSubcoreMesh(core_axis_name="c", subcore_axis_name="s"),
)(_sc_gather)


@jax.jit
def kernel(x):
    x4 = x.reshape(B, K, S, L)
    out = _gather_call(x4)
    return out.reshape(B, K, L)


# speedup vs baseline: 10.5361x; 10.5361x over previous
import functools

import jax
import jax.numpy as jnp
from jax import lax
from jax.experimental import pallas as pl
from jax.experimental.pallas import tpu as pltpu
from jax.experimental.pallas import tpu_sc as plsc

B, C, L = 4, 1024, 8192
K = 128
S = C // K
NW = 32
ROWS_PER_W = (B * K) // NW
CHUNK = 4
NCHUNK = ROWS_PER_W // CHUNK
NBUF = 2


def _sc_gather(x_hbm, out_hbm, buf, g_sem, w_sem):
    wid = lax.axis_index("s") * 2 + lax.axis_index("c")
    r0 = wid * ROWS_PER_W

    def gather(i, slot):
        return pltpu.async_copy(
            x_hbm.at[pl.ds(r0 + i * CHUNK, CHUNK), pl.ds(0, 1), :],
            buf.at[slot],
            g_sem.at[slot],
        )

    def write(i, slot):
        return pltpu.async_copy(
            buf.at[slot],
            out_hbm.at[pl.ds(r0 + i * CHUNK, CHUNK), :, :],
            w_sem.at[slot],
        )

    pending_g = [None] * NBUF
    prev_writes = [None] * NBUF
    pending_g[0] = gather(0, 0)
    for i in range(NCHUNK):
        slot = i % NBUF
        nslot = (i + 1) % NBUF
        if i + 1 < NCHUNK:
            if prev_writes[nslot] is not None:
                prev_writes[nslot].wait()
                prev_writes[nslot] = None
            pending_g[nslot] = gather(i + 1, nslot)
        pending_g[slot].wait()
        prev_writes[slot] = write(i, slot)
    for w in prev_writes:
        if w is not None:
            w.wait()


_gather_call = functools.partial(
    pl.kernel,
    out_type=jax.ShapeDtypeStruct((B * K, 1, L), jnp.float32),
    mesh=plsc.VectorSubcoreMesh(core_axis_name="c", subcore_axis_name="s"),
    scratch_types=[
        pltpu.VMEM((NBUF, CHUNK, 1, L), jnp.float32),
        pltpu.SemaphoreType.DMA((NBUF,)),
        pltpu.SemaphoreType.DMA((NBUF,)),
    ],
)(_sc_gather)


@jax.jit
def kernel(x):
    x3 = x.reshape(B * K, S, L)
    out = _gather_call(x3)
    return out.reshape(B, K, L)
